# baseline (speedup 1.0000x reference)
import jax
import jax.numpy as jnp
from jax import lax
from jax.experimental import pallas as pl
from jax.experimental.pallas import tpu as pltpu
from jax.experimental.pallas import tpu_sc as plsc

_N = 10000
_E = 320000
_D = 128
_H = 64
_R = 2
_B = 30

_NC = 2
_NS = 16
_NW = _NC * _NS
_EPW = _E // _NW
_CH = 80
_NCH = _EPW // _CH
_VPC = _CH // 16
_RN = _R * _N


def _weights_body(c1, b1, c2, b2, w1f, w2f):
    w1f[...] = jnp.dot(c1[...], b1[...], preferred_element_type=jnp.float32)
    w2f[...] = jnp.dot(c2[...], b2[...], preferred_element_type=jnp.float32)


def _weights(c1, b1f, c2, b2f):
    return pl.pallas_call(
        _weights_body,
        out_shape=(
            jax.ShapeDtypeStruct((_R, _D * _H), jnp.float32),
            jax.ShapeDtypeStruct((_R, _H * _H), jnp.float32),
        ),
    )(c1, b1f, c2, b2f)


def _dense1_body(w1, x, r1, t1, xr1):
    xv = x[...]
    t1[0] = jnp.dot(xv, w1[0], preferred_element_type=jnp.float32)
    t1[1] = jnp.dot(xv, w1[1], preferred_element_type=jnp.float32)
    xr1[...] = jnp.dot(xv, r1[...], preferred_element_type=jnp.float32)


def _dense1(w1, x, root1):
    return pl.pallas_call(
        _dense1_body,
        out_shape=(
            jax.ShapeDtypeStruct((_R, _N, _H), jnp.float32),
            jax.ShapeDtypeStruct((_N, _H), jnp.float32),
        ),
    )(w1, x, root1)


def _edges_body_factory(with_cnt):
    def body(t_hbm, src_hbm, dst_hbm, et_hbm, z2d_hbm, *rest):
        if with_cnt:
            (z1d_hbm, s_out, cnt_out, srcv, dstv, etv, gidx, qidx,
             rows0, rows1, rows2, ones_v,
             gs0, gs1, gs2, ss0, ss1, ss2, s_sh, cnt_sh) = rest
        else:
            (s_out, srcv, dstv, etv, gidx, qidx,
             rows0, rows1, rows2, ones_v,
             gs0, gs1, gs2, ss0, ss1, ss2, s_sh) = rest
        rows = (rows0, rows1, rows2)
        gsem = (gs0, gs1, gs2)
        ssem = (ss0, ss1, ss2)
        cid = lax.axis_index("c")
        sid = lax.axis_index("s")
        wid = cid * _NS + sid
        base = wid * _EPW

        @pl.when(sid < 10)
        def _():
            pltpu.sync_copy(z2d_hbm, s_sh.at[pl.ds(sid * 2000, 2000)])
            if with_cnt:
                pltpu.sync_copy(z1d_hbm, cnt_sh.at[pl.ds(sid * 2000, 2000)])

        pltpu.sync_copy(src_hbm.at[pl.ds(base, _EPW)], srcv)
        pltpu.sync_copy(dst_hbm.at[pl.ds(base, _EPW)], dstv)
        pltpu.sync_copy(et_hbm.at[pl.ds(base, _EPW)], etv)

        for k in range(_VPC):
            ones_v[pl.ds(k * 16, 16)] = jnp.ones((16,), jnp.float32)

        def build_g(c, slot):
            for k in range(_VPC):
                sl = pl.ds(c * _CH + k * 16, 16)
                gidx[slot, pl.ds(k * 16, 16)] = etv[sl] * _N + srcv[sl]

        def build_q(c, slot):
            for k in range(_VPC):
                sl = pl.ds(c * _CH + k * 16, 16)
                qidx[slot, pl.ds(k * 16, 16)] = etv[sl] * _N + dstv[sl]

        def g_start(c, slot):
            build_g(c, slot)
            pltpu.async_copy(t_hbm.at[gidx.at[slot]], rows[slot], gsem[slot])

        def g_wait(slot):
            pltpu.make_async_copy(t_hbm.at[gidx.at[slot]], rows[slot],
                                  gsem[slot]).wait()

        def s_start(c, slot):
            build_q(c, slot)
            pltpu.async_copy(rows[slot], s_sh.at[qidx.at[slot]], ssem[slot],
                             add=True)
            if with_cnt:
                pltpu.async_copy(ones_v, cnt_sh.at[qidx.at[slot]],
                                 ssem[slot], add=True)

        def s_wait(slot):
            pltpu.make_async_copy(rows[slot], s_sh.at[qidx.at[slot]],
                                  ssem[slot]).wait()
            if with_cnt:
                pltpu.make_async_copy(ones_v, cnt_sh.at[qidx.at[slot]],
                                      ssem[slot]).wait()

        plsc.subcore_barrier()

        g_start(0, 0)
        g_start(1, 1)

        def consume(c, slot, nxt_slot):
            g_wait(slot)
            s_start(c, slot)

            @pl.when(c >= 1)
            def _():
                s_wait(nxt_slot)

            @pl.when(c + 2 < _NCH)
            def _():
                g_start(c + 2, nxt_slot)

        def triple(t, carry):
            c0 = 3 * t
            consume(c0, 0, 2)
            consume(c0 + 1, 1, 0)
            consume(c0 + 2, 2, 1)
            return carry

        lax.fori_loop(0, _NCH // 3, triple, 0)
        consume(_NCH - 2, 0, 2)
        consume(_NCH - 1, 1, 0)
        s_wait(1)

        plsc.subcore_barrier()

        @pl.when(sid < 10)
        def _():
            pltpu.sync_copy(s_sh.at[pl.ds(sid * 2000, 2000)],
                            s_out.at[cid, pl.ds(sid * 2000, 2000)])
            if with_cnt:
                pltpu.sync_copy(cnt_sh.at[pl.ds(sid * 2000, 2000)],
                                cnt_out.at[pl.ds(cid * _RN + sid * 2000,
                                                 2000)])

    return body


def _edges(tflat, src, dst, et, z2d, z1d, with_cnt):
    mesh = plsc.VectorSubcoreMesh(core_axis_name="c", subcore_axis_name="s")
    if with_cnt:
        out_type = (
            jax.ShapeDtypeStruct((_NC, _RN, _H), jnp.float32),
            jax.ShapeDtypeStruct((_NC * _RN,), jnp.float32),
        )
    else:
        out_type = jax.ShapeDtypeStruct((_NC, _RN, _H), jnp.float32)
    scratch = [
        pltpu.VMEM((_EPW,), jnp.int32),
        pltpu.VMEM((_EPW,), jnp.int32),
        pltpu.VMEM((_EPW,), jnp.int32),
        pltpu.VMEM((3, _CH), jnp.int32),
        pltpu.VMEM((3, _CH), jnp.int32),
        pltpu.VMEM((_CH, _H), jnp.float32),
        pltpu.VMEM((_CH, _H), jnp.float32),
        pltpu.VMEM((_CH, _H), jnp.float32),
        pltpu.VMEM((_CH,), jnp.float32),
        pltpu.SemaphoreType.DMA,
        pltpu.SemaphoreType.DMA,
        pltpu.SemaphoreType.DMA,
        pltpu.SemaphoreType.DMA,
        pltpu.SemaphoreType.DMA,
        pltpu.SemaphoreType.DMA,
        pltpu.VMEM_SHARED((_RN, _H), jnp.float32),
    ]
    if with_cnt:
        scratch.append(pltpu.VMEM_SHARED((_RN,), jnp.float32))
    f = pl.kernel(
        _edges_body_factory(with_cnt),
        out_type=out_type,
        mesh=mesh,
        scratch_types=scratch,
        compiler_params=pltpu.CompilerParams(use_tc_tiling_on_sc=False),
    )
    if with_cnt:
        return f(tflat, src, dst, et, z2d, z1d)
    return f(tflat, src, dst, et, z2d)


def _combine1_body(sp, cp, xr, b1, w2, r2, t2, xr2o):
    c0 = cp[0, 0] + cp[1, 0]
    c1 = cp[0, 1] + cp[1, 1]
    n0 = (1.0 / jnp.maximum(c0, 1.0))[:, None]
    n1 = (1.0 / jnp.maximum(c1, 1.0))[:, None]
    s0 = sp[0, 0] + sp[1, 0]
    s1 = sp[0, 1] + sp[1, 1]
    out1 = jnp.maximum(
        s0 * n0 + s1 * n1 + xr[...] + b1[...][None, :], 0.0)
    t2[0] = jnp.dot(out1, w2[0], preferred_element_type=jnp.float32)
    t2[1] = jnp.dot(out1, w2[1], preferred_element_type=jnp.float32)
    xr2o[...] = jnp.dot(out1, r2[...], preferred_element_type=jnp.float32)


def _combine1(sp, cp, xr1, bias1, w2, root2):
    return pl.pallas_call(
        _combine1_body,
        out_shape=(
            jax.ShapeDtypeStruct((_R, _N, _H), jnp.float32),
            jax.ShapeDtypeStruct((_N, _H), jnp.float32),
        ),
    )(sp, cp, xr1, bias1, w2, root2)


def _combine2_body(sp, cp, xr, b2, wg, ats, atd, ho, co):
    c0 = cp[0, 0] + cp[1, 0]
    c1 = cp[0, 1] + cp[1, 1]
    n0 = (1.0 / jnp.maximum(c0, 1.0))[:, None]
    n1 = (1.0 / jnp.maximum(c1, 1.0))[:, None]
    s0 = sp[0, 0] + sp[1, 0]
    s1 = sp[0, 1] + sp[1, 1]
    out2 = s0 * n0 + s1 * n1 + xr[...] + b2[...][None, :]
    h = jnp.dot(out2, wg[...], preferred_element_type=jnp.float32)
    m = jnp.max(h * ats[...])
    ho[...] = h
    co[...] = jnp.concatenate(
        [jnp.broadcast_to(ats[...], (1, 128)),
         jnp.broadcast_to(atd[...], (1, 128)),
         jnp.broadcast_to(m, (1, 128))], axis=0)


def _combine2(sp, cp, xr2, bias2, wg, ats, atd):
    return pl.pallas_call(
        _combine2_body,
        out_shape=(
            jax.ShapeDtypeStruct((_N, 1), jnp.float32),
            jax.ShapeDtypeStruct((3, 128), jnp.float32),
        ),
    )(sp, cp, xr2, bias2, wg, ats, atd)


def _gat_body(h_hbm, c_hbm, src_hbm, dst_hbm, z1d_hbm,
              gpart,
              hv, constv, srcv, dstv, didx, pbuf, phbuf,
              ps0, ps1, den_sh, num_sh):
    psem = (ps0, ps1)
    cid = lax.axis_index("c")
    sid = lax.axis_index("s")
    wid = cid * _NS + sid
    base = wid * _EPW

    @pl.when(sid < 5)
    def _():
        pltpu.sync_copy(z1d_hbm, den_sh.at[pl.ds(sid * 2000, 2000)])

    @pl.when(jnp.logical_and(sid >= 5, sid < 10))
    def _():
        pltpu.sync_copy(z1d_hbm, num_sh.at[pl.ds((sid - 5) * 2000, 2000)])

    pltpu.sync_copy(h_hbm, hv)
    pltpu.sync_copy(c_hbm, constv)
    pltpu.sync_copy(src_hbm.at[pl.ds(base, _EPW)], srcv)
    pltpu.sync_copy(dst_hbm.at[pl.ds(base, _EPW)], dstv)

    plsc.subcore_barrier()

    ats16 = constv[0, pl.ds(0, 16)]
    atd16 = constv[1, pl.ds(0, 16)]
    m16 = constv[2, pl.ds(0, 16)]

    def compute(c, slot):
        for k in range(_VPC):
            sl = pl.ds(c * _CH + k * 16, 16)
            s16 = srcv[sl]
            d16 = dstv[sl]
            hs = plsc.load_gather(hv, [s16])
            hd = plsc.load_gather(hv, [d16])
            a_d = atd16 * hd
            t = ats16 * hs + a_d
            zd = a_d + m16
            cd = jnp.maximum(zd, 0.2 * zd)
            p = jnp.exp(jnp.maximum(t, 0.2 * t) - cd)
            ksl = pl.ds(k * 16, 16)
            pbuf[slot, ksl] = p
            phbuf[slot, ksl] = p * hs
            didx[slot, ksl] = d16

    def s_start(slot):
        pltpu.async_copy(pbuf.at[slot], den_sh.at[didx.at[slot]],
                         psem[slot], add=True)
        pltpu.async_copy(phbuf.at[slot], num_sh.at[didx.at[slot]],
                         psem[slot], add=True)

    def s_wait(slot):
        pltpu.make_async_copy(pbuf.at[slot], den_sh.at[didx.at[slot]],
                              psem[slot]).wait()
        pltpu.make_async_copy(phbuf.at[slot], num_sh.at[didx.at[slot]],
                              psem[slot]).wait()

    compute(0, 0)
    s_start(0)
    compute(1, 1)
    s_start(1)

    def pair(p, carry):
        c0 = 2 * p
        c1 = c0 + 1
        s_wait(0)
        compute(c0, 0)
        s_start(0)

        @pl.when(c1 < _NCH)
        def _():
            s_wait(1)
            compute(c1, 1)
            s_start(1)

        return carry

    lax.fori_loop(1, (_NCH + 1) // 2, pair, 0)
    s_wait(0)
    s_wait(1)

    plsc.subcore_barrier()

    @pl.when(sid < 5)
    def _():
        pltpu.sync_copy(den_sh.at[pl.ds(sid * 2000, 2000)],
                        gpart.at[pl.ds(cid * 2 * _N + sid * 2000, 2000)])

    @pl.when(jnp.logical_and(sid >= 5, sid < 10))
    def _():
        pltpu.sync_copy(num_sh.at[pl.ds((sid - 5) * 2000, 2000)],
                        gpart.at[pl.ds(cid * 2 * _N + _N + (sid - 5) * 2000,
                                       2000)])


def _gat(h, consts, src, dst, z1d):
    mesh = plsc.VectorSubcoreMesh(core_axis_name="c", subcore_axis_name="s")
    f = pl.kernel(
        _gat_body,
        out_type=jax.ShapeDtypeStruct((_NC * 2 * _N,), jnp.float32),
        mesh=mesh,
        scratch_types=[
            pltpu.VMEM((_N,), jnp.float32),
            pltpu.VMEM((3, 128), jnp.float32),
            pltpu.VMEM((_EPW,), jnp.int32),
            pltpu.VMEM((_EPW,), jnp.int32),
            pltpu.VMEM((2, _CH), jnp.int32),
            pltpu.VMEM((2, _CH), jnp.float32),
            pltpu.VMEM((2, _CH), jnp.float32),
            pltpu.SemaphoreType.DMA,
            pltpu.SemaphoreType.DMA,
            pltpu.VMEM_SHARED((_N,), jnp.float32),
            pltpu.VMEM_SHARED((_N,), jnp.float32),
        ],
        compiler_params=pltpu.CompilerParams(use_tc_tiling_on_sc=False,
                                             needs_layout_passes=False),
    )
    return f(h, consts, src, dst, z1d)


def _final_body(gp, hh, ats, atd, bg, out):
    h = hh[...]
    a_s = h * ats[...]
    a_d = h * atd[...]
    m = jnp.max(a_s)
    zd = a_d + m
    cd = jnp.maximum(zd, 0.2 * zd)
    t = a_s + a_d
    ps = jnp.exp(jnp.maximum(t, 0.2 * t) - cd)
    d = gp[0, 0] + gp[1, 0] + ps
    nu = gp[0, 1] + gp[1, 1] + ps * h
    out[...] = nu / jnp.maximum(d, 1e-30) + bg[...]


def _final(gp, hh, ats, atd, bg):
    return pl.pallas_call(
        _final_body,
        out_shape=jax.ShapeDtypeStruct((1, _N), jnp.float32),
    )(gp, hh, ats, atd, bg)


def kernel(x, edge_index, edge_types, bases1, comp1, root1, bias1,
           bases2, comp2, root2, bias2, w_gat, att_src, att_dst, bias_gat):
    src = edge_index[0]
    dst = edge_index[1]
    et = edge_types

    w1f, w2f = _weights(comp1, bases1.reshape(_B, _D * _H),
                        comp2, bases2.reshape(_B, _H * _H))
    t1, xr1 = _dense1(w1f.reshape(_R, _D, _H), x, root1)

    z2d = jnp.zeros((2000, _H), jnp.float32)
    z1d = jnp.zeros((2000,), jnp.float32)

    s1p, c1p = _edges(t1.reshape(_RN, _H), src, dst, et, z2d, z1d,
                      with_cnt=True)
    sp1 = s1p.reshape(_NC, _R, _N, _H)
    cp1 = c1p.reshape(_NC, _R, _N)

    t2, xr2 = _combine1(sp1, cp1, xr1, bias1,
                        w2f.reshape(_R, _H, _H), root2)

    s2p = _edges(t2.reshape(_RN, _H), src, dst, et, z2d, z1d,
                 with_cnt=False)
    sp2 = s2p.reshape(_NC, _R, _N, _H)

    h, consts = _combine2(sp2, cp1, xr2, bias2, w_gat, att_src, att_dst)

    gp = _gat(h.reshape(_N), consts, src, dst, z1d)

    out = _final(gp.reshape(_NC, 2, 1, _N), h.reshape(1, _N),
                 att_src, att_dst, bias_gat)
    return out.reshape(_N, 1)


# speedup vs baseline: 1.0137x; 1.0137x over previous
import jax
import jax.numpy as jnp
from jax import lax
from jax.experimental import pallas as pl
from jax.experimental.pallas import tpu as pltpu
from jax.experimental.pallas import tpu_sc as plsc

_N = 10000
_E = 320000
_D = 128
_H = 64
_R = 2
_B = 30

_NC = 2
_NS = 16
_NW = _NC * _NS
_EPW = _E // _NW
_CH = 80
_NCH = _EPW // _CH
_VPC = _CH // 16
_RN = _R * _N


def _kron_eye(crow):
    eye = (lax.broadcasted_iota(jnp.int32, (_H, _H), 0) ==
           lax.broadcasted_iota(jnp.int32, (_H, _H), 1)).astype(jnp.float32)
    return (crow[:, None, None] * eye[None, :, :]).reshape(_B * _H, _H)


def _dense1_body(c1, bt1, x, r1, t1, xr1):
    w0 = jnp.dot(bt1[...], _kron_eye(c1[0]),
                 preferred_element_type=jnp.float32)
    w1 = jnp.dot(bt1[...], _kron_eye(c1[1]),
                 preferred_element_type=jnp.float32)
    xv = x[...]
    t1[0] = jnp.dot(xv, w0, preferred_element_type=jnp.float32)
    t1[1] = jnp.dot(xv, w1, preferred_element_type=jnp.float32)
    xr1[...] = jnp.dot(xv, r1[...], preferred_element_type=jnp.float32)


def _dense1(c1, bt1, x, root1):
    return pl.pallas_call(
        _dense1_body,
        out_shape=(
            jax.ShapeDtypeStruct((_R, _N, _H), jnp.float32),
            jax.ShapeDtypeStruct((_N, _H), jnp.float32),
        ),
    )(c1, bt1, x, root1)


def _edges_body_factory(with_cnt):
    def body(t_hbm, src_hbm, dst_hbm, et_hbm, z2d_hbm, *rest):
        if with_cnt:
            (z1d_hbm, s_out, cnt_out, srcv, dstv, etv, gidx, qidx,
             rows0, rows1, rows2, ones_v,
             gs0, gs1, gs2, ss0, ss1, ss2, s_sh, cnt_sh) = rest
        else:
            (s_out, srcv, dstv, etv, gidx, qidx,
             rows0, rows1, rows2, ones_v,
             gs0, gs1, gs2, ss0, ss1, ss2, s_sh) = rest
        rows = (rows0, rows1, rows2)
        gsem = (gs0, gs1, gs2)
        ssem = (ss0, ss1, ss2)
        cid = lax.axis_index("c")
        sid = lax.axis_index("s")
        wid = cid * _NS + sid
        base = wid * _EPW

        @pl.when(sid < 10)
        def _():
            pltpu.sync_copy(z2d_hbm, s_sh.at[pl.ds(sid * 2000, 2000)])
            if with_cnt:
                pltpu.sync_copy(z1d_hbm, cnt_sh.at[pl.ds(sid * 2000, 2000)])

        pltpu.sync_copy(src_hbm.at[pl.ds(base, _EPW)], srcv)
        pltpu.sync_copy(dst_hbm.at[pl.ds(base, _EPW)], dstv)
        pltpu.sync_copy(et_hbm.at[pl.ds(base, _EPW)], etv)

        for k in range(_VPC):
            ones_v[pl.ds(k * 16, 16)] = jnp.ones((16,), jnp.float32)

        def build_g(c, slot):
            for k in range(_VPC):
                sl = pl.ds(c * _CH + k * 16, 16)
                gidx[slot, pl.ds(k * 16, 16)] = etv[sl] * _N + srcv[sl]

        def build_q(c, slot):
            for k in range(_VPC):
                sl = pl.ds(c * _CH + k * 16, 16)
                qidx[slot, pl.ds(k * 16, 16)] = etv[sl] * _N + dstv[sl]

        def g_start(c, slot):
            build_g(c, slot)
            pltpu.async_copy(t_hbm.at[gidx.at[slot]], rows[slot], gsem[slot])

        def g_wait(slot):
            pltpu.make_async_copy(t_hbm.at[gidx.at[slot]], rows[slot],
                                  gsem[slot]).wait()

        def s_start(c, slot):
            build_q(c, slot)
            pltpu.async_copy(rows[slot], s_sh.at[qidx.at[slot]], ssem[slot],
                             add=True)
            if with_cnt:
                pltpu.async_copy(ones_v, cnt_sh.at[qidx.at[slot]],
                                 ssem[slot], add=True)

        def s_wait(slot):
            pltpu.make_async_copy(rows[slot], s_sh.at[qidx.at[slot]],
                                  ssem[slot]).wait()
            if with_cnt:
                pltpu.make_async_copy(ones_v, cnt_sh.at[qidx.at[slot]],
                                      ssem[slot]).wait()

        plsc.subcore_barrier()

        g_start(0, 0)
        g_start(1, 1)

        def consume(c, slot, nxt_slot):
            g_wait(slot)
            s_start(c, slot)

            @pl.when(c >= 1)
            def _():
                s_wait(nxt_slot)

            @pl.when(c + 2 < _NCH)
            def _():
                g_start(c + 2, nxt_slot)

        def triple(t, carry):
            c0 = 3 * t
            consume(c0, 0, 2)
            consume(c0 + 1, 1, 0)
            consume(c0 + 2, 2, 1)
            return carry

        lax.fori_loop(0, _NCH // 3, triple, 0)
        consume(_NCH - 2, 0, 2)
        consume(_NCH - 1, 1, 0)
        s_wait(1)

        plsc.subcore_barrier()

        @pl.when(sid < 10)
        def _():
            pltpu.sync_copy(s_sh.at[pl.ds(sid * 2000, 2000)],
                            s_out.at[cid, pl.ds(sid * 2000, 2000)])
            if with_cnt:
                pltpu.sync_copy(cnt_sh.at[pl.ds(sid * 2000, 2000)],
                                cnt_out.at[pl.ds(cid * _RN + sid * 2000,
                                                 2000)])

    return body


def _edges(tflat, src, dst, et, z2d, z1d, with_cnt):
    mesh = plsc.VectorSubcoreMesh(core_axis_name="c", subcore_axis_name="s")
    if with_cnt:
        out_type = (
            jax.ShapeDtypeStruct((_NC, _RN, _H), jnp.float32),
            jax.ShapeDtypeStruct((_NC * _RN,), jnp.float32),
        )
    else:
        out_type = jax.ShapeDtypeStruct((_NC, _RN, _H), jnp.float32)
    scratch = [
        pltpu.VMEM((_EPW,), jnp.int32),
        pltpu.VMEM((_EPW,), jnp.int32),
        pltpu.VMEM((_EPW,), jnp.int32),
        pltpu.VMEM((3, _CH), jnp.int32),
        pltpu.VMEM((3, _CH), jnp.int32),
        pltpu.VMEM((_CH, _H), jnp.float32),
        pltpu.VMEM((_CH, _H), jnp.float32),
        pltpu.VMEM((_CH, _H), jnp.float32),
        pltpu.VMEM((_CH,), jnp.float32),
        pltpu.SemaphoreType.DMA,
        pltpu.SemaphoreType.DMA,
        pltpu.SemaphoreType.DMA,
        pltpu.SemaphoreType.DMA,
        pltpu.SemaphoreType.DMA,
        pltpu.SemaphoreType.DMA,
        pltpu.VMEM_SHARED((_RN, _H), jnp.float32),
    ]
    if with_cnt:
        scratch.append(pltpu.VMEM_SHARED((_RN,), jnp.float32))
    f = pl.kernel(
        _edges_body_factory(with_cnt),
        out_type=out_type,
        mesh=mesh,
        scratch_types=scratch,
        compiler_params=pltpu.CompilerParams(use_tc_tiling_on_sc=False),
    )
    if with_cnt:
        return f(tflat, src, dst, et, z2d, z1d)
    return f(tflat, src, dst, et, z2d)


def _combine1_body(sp, cp, xr, b1, c2, bt2, r2, t2, xr2o):
    c0 = cp[0, 0] + cp[1, 0]
    c1 = cp[0, 1] + cp[1, 1]
    n0 = (1.0 / jnp.maximum(c0, 1.0))[:, None]
    n1 = (1.0 / jnp.maximum(c1, 1.0))[:, None]
    s0 = sp[0, 0] + sp[1, 0]
    s1 = sp[0, 1] + sp[1, 1]
    out1 = jnp.maximum(
        s0 * n0 + s1 * n1 + xr[...] + b1[...][None, :], 0.0)
    w20 = jnp.dot(bt2[...], _kron_eye(c2[0]),
                  preferred_element_type=jnp.float32)
    w21 = jnp.dot(bt2[...], _kron_eye(c2[1]),
                  preferred_element_type=jnp.float32)
    t2[0] = jnp.dot(out1, w20, preferred_element_type=jnp.float32)
    t2[1] = jnp.dot(out1, w21, preferred_element_type=jnp.float32)
    xr2o[...] = jnp.dot(out1, r2[...], preferred_element_type=jnp.float32)


def _combine1(sp, cp, xr1, bias1, c2, bt2, root2):
    return pl.pallas_call(
        _combine1_body,
        out_shape=(
            jax.ShapeDtypeStruct((_R, _N, _H), jnp.float32),
            jax.ShapeDtypeStruct((_N, _H), jnp.float32),
        ),
    )(sp, cp, xr1, bias1, c2, bt2, root2)


def _combine2_body(sp, cp, xr, b2, wg, ats, atd, ho, co):
    c0 = cp[0, 0] + cp[1, 0]
    c1 = cp[0, 1] + cp[1, 1]
    n0 = (1.0 / jnp.maximum(c0, 1.0))[:, None]
    n1 = (1.0 / jnp.maximum(c1, 1.0))[:, None]
    s0 = sp[0, 0] + sp[1, 0]
    s1 = sp[0, 1] + sp[1, 1]
    out2 = s0 * n0 + s1 * n1 + xr[...] + b2[...][None, :]
    h = jnp.dot(out2, wg[...], preferred_element_type=jnp.float32)
    m = jnp.max(h * ats[...])
    ho[...] = h
    co[...] = jnp.concatenate(
        [jnp.broadcast_to(ats[...], (1, 128)),
         jnp.broadcast_to(atd[...], (1, 128)),
         jnp.broadcast_to(m, (1, 128))], axis=0)


def _combine2(sp, cp, xr2, bias2, wg, ats, atd):
    return pl.pallas_call(
        _combine2_body,
        out_shape=(
            jax.ShapeDtypeStruct((_N, 1), jnp.float32),
            jax.ShapeDtypeStruct((3, 128), jnp.float32),
        ),
    )(sp, cp, xr2, bias2, wg, ats, atd)


def _gat_body(h_hbm, c_hbm, src_hbm, dst_hbm, z1d_hbm,
              gpart,
              hv, constv, srcv, dstv, didx, pbuf, phbuf,
              ps0, ps1, den_sh, num_sh):
    psem = (ps0, ps1)
    cid = lax.axis_index("c")
    sid = lax.axis_index("s")
    wid = cid * _NS + sid
    base = wid * _EPW

    @pl.when(sid < 5)
    def _():
        pltpu.sync_copy(z1d_hbm, den_sh.at[pl.ds(sid * 2000, 2000)])

    @pl.when(jnp.logical_and(sid >= 5, sid < 10))
    def _():
        pltpu.sync_copy(z1d_hbm, num_sh.at[pl.ds((sid - 5) * 2000, 2000)])

    pltpu.sync_copy(h_hbm, hv)
    pltpu.sync_copy(c_hbm, constv)
    pltpu.sync_copy(src_hbm.at[pl.ds(base, _EPW)], srcv)
    pltpu.sync_copy(dst_hbm.at[pl.ds(base, _EPW)], dstv)

    plsc.subcore_barrier()

    ats16 = constv[0, pl.ds(0, 16)]
    atd16 = constv[1, pl.ds(0, 16)]
    m16 = constv[2, pl.ds(0, 16)]

    def compute(c, slot):
        for k in range(_VPC):
            sl = pl.ds(c * _CH + k * 16, 16)
            s16 = srcv[sl]
            d16 = dstv[sl]
            hs = plsc.load_gather(hv, [s16])
            hd = plsc.load_gather(hv, [d16])
            a_d = atd16 * hd
            t = ats16 * hs + a_d
            zd = a_d + m16
            cd = jnp.maximum(zd, 0.2 * zd)
            p = jnp.exp(jnp.maximum(t, 0.2 * t) - cd)
            ksl = pl.ds(k * 16, 16)
            pbuf[slot, ksl] = p
            phbuf[slot, ksl] = p * hs
            didx[slot, ksl] = d16

    def s_start(slot):
        pltpu.async_copy(pbuf.at[slot], den_sh.at[didx.at[slot]],
                         psem[slot], add=True)
        pltpu.async_copy(phbuf.at[slot], num_sh.at[didx.at[slot]],
                         psem[slot], add=True)

    def s_wait(slot):
        pltpu.make_async_copy(pbuf.at[slot], den_sh.at[didx.at[slot]],
                              psem[slot]).wait()
        pltpu.make_async_copy(phbuf.at[slot], num_sh.at[didx.at[slot]],
                              psem[slot]).wait()

    compute(0, 0)
    s_start(0)
    compute(1, 1)
    s_start(1)

    def pair(p, carry):
        c0 = 2 * p
        c1 = c0 + 1
        s_wait(0)
        compute(c0, 0)
        s_start(0)

        @pl.when(c1 < _NCH)
        def _():
            s_wait(1)
            compute(c1, 1)
            s_start(1)

        return carry

    lax.fori_loop(1, (_NCH + 1) // 2, pair, 0)
    s_wait(0)
    s_wait(1)

    plsc.subcore_barrier()

    @pl.when(sid < 5)
    def _():
        pltpu.sync_copy(den_sh.at[pl.ds(sid * 2000, 2000)],
                        gpart.at[pl.ds(cid * 2 * _N + sid * 2000, 2000)])

    @pl.when(jnp.logical_and(sid >= 5, sid < 10))
    def _():
        pltpu.sync_copy(num_sh.at[pl.ds((sid - 5) * 2000, 2000)],
                        gpart.at[pl.ds(cid * 2 * _N + _N + (sid - 5) * 2000,
                                       2000)])


def _gat(h, consts, src, dst, z1d):
    mesh = plsc.VectorSubcoreMesh(core_axis_name="c", subcore_axis_name="s")
    f = pl.kernel(
        _gat_body,
        out_type=jax.ShapeDtypeStruct((_NC * 2 * _N,), jnp.float32),
        mesh=mesh,
        scratch_types=[
            pltpu.VMEM((_N,), jnp.float32),
            pltpu.VMEM((3, 128), jnp.float32),
            pltpu.VMEM((_EPW,), jnp.int32),
            pltpu.VMEM((_EPW,), jnp.int32),
            pltpu.VMEM((2, _CH), jnp.int32),
            pltpu.VMEM((2, _CH), jnp.float32),
            pltpu.VMEM((2, _CH), jnp.float32),
            pltpu.SemaphoreType.DMA,
            pltpu.SemaphoreType.DMA,
            pltpu.VMEM_SHARED((_N,), jnp.float32),
            pltpu.VMEM_SHARED((_N,), jnp.float32),
        ],
        compiler_params=pltpu.CompilerParams(use_tc_tiling_on_sc=False,
                                             needs_layout_passes=False),
    )
    return f(h, consts, src, dst, z1d)


def _final_body(gp, hh, ats, atd, bg, out):
    h = hh[...]
    a_s = h * ats[...]
    a_d = h * atd[...]
    m = jnp.max(a_s)
    zd = a_d + m
    cd = jnp.maximum(zd, 0.2 * zd)
    t = a_s + a_d
    ps = jnp.exp(jnp.maximum(t, 0.2 * t) - cd)
    d = gp[0, 0] + gp[1, 0] + ps
    nu = gp[0, 1] + gp[1, 1] + ps * h
    out[...] = nu / jnp.maximum(d, 1e-30) + bg[...]


def _final(gp, hh, ats, atd, bg):
    return pl.pallas_call(
        _final_body,
        out_shape=jax.ShapeDtypeStruct((1, _N), jnp.float32),
    )(gp, hh, ats, atd, bg)


def kernel(x, edge_index, edge_types, bases1, comp1, root1, bias1,
           bases2, comp2, root2, bias2, w_gat, att_src, att_dst, bias_gat):
    src = edge_index[0]
    dst = edge_index[1]
    et = edge_types

    bt1 = bases1.transpose(1, 0, 2).reshape(_D, _B * _H)
    bt2 = bases2.transpose(1, 0, 2).reshape(_H, _B * _H)
    t1, xr1 = _dense1(comp1, bt1, x, root1)

    z2d = jnp.zeros((2000, _H), jnp.float32)
    z1d = jnp.zeros((2000,), jnp.float32)

    s1p, c1p = _edges(t1.reshape(_RN, _H), src, dst, et, z2d, z1d,
                      with_cnt=True)
    sp1 = s1p.reshape(_NC, _R, _N, _H)
    cp1 = c1p.reshape(_NC, _R, _N)

    t2, xr2 = _combine1(sp1, cp1, xr1, bias1, comp2, bt2, root2)

    s2p = _edges(t2.reshape(_RN, _H), src, dst, et, z2d, z1d,
                 with_cnt=False)
    sp2 = s2p.reshape(_NC, _R, _N, _H)

    h, consts = _combine2(sp2, cp1, xr2, bias2, w_gat, att_src, att_dst)

    gp = _gat(h.reshape(_N), consts, src, dst, z1d)

    out = _final(gp.reshape(_NC, 2, 1, _N), h.reshape(1, _N),
                 att_src, att_dst, bias_gat)
    return out.reshape(_N, 1)
